# baseline (speedup 1.0000x reference)
import functools

import jax
import jax.numpy as jnp
from jax import lax
from jax.experimental import pallas as pl
from jax.experimental.pallas import tpu as pltpu
from jax.experimental.pallas import tpu_sc as plsc

N = 10000
D = 128
E = 320000

NC = 2
NS = 16
NW = NC * NS
EPW = E // NW
CH = 128
NCH = 80
PAD = NCH * CH - EPW
NPAD = N + NS
RPT = N // NS
DEGL = 16


def _zero_rows(buf, nrows, ncols, dtype=jnp.float32):
  lanes = 32 if dtype == jnp.bfloat16 else 16
  z = jnp.zeros((lanes,), dtype)

  def body(r, _):
    for j in range(ncols // lanes):
      buf[r, pl.ds(j * lanes, lanes)] = z
    return 0

  lax.fori_loop(0, nrows, body, 0)


def _sc_mesh():
  return plsc.VectorSubcoreMesh(
      core_axis_name="c", subcore_axis_name="s", num_cores=NC, num_subcores=NS
  )


def _deg_body(ei_hbm, deg_out, idx_v, ones_v, acc_sh, sem):
  c = lax.axis_index("c")
  s = lax.axis_index("s")
  w = c * NS + s

  _zero_rows(ones_v, RPT, DEGL)
  pltpu.sync_copy(ones_v, acc_sh.at[pl.ds(s * RPT, RPT)])
  one = jnp.full((16,), 1.0, jnp.float32)

  def fill(r, _):
    ones_v[r, :] = one
    return 0

  lax.fori_loop(0, CH, fill, 0)
  pltpu.sync_copy(ei_hbm.at[1, w], idx_v)
  plsc.subcore_barrier()

  def chunk(g, _):
    pltpu.async_copy(
        ones_v.at[pl.ds(0, CH)], acc_sh.at[idx_v.at[g]], sem, add=True
    )
    return 0

  lax.fori_loop(0, NCH, chunk, 0)

  def drain(g, _):
    pltpu.make_async_copy(
        ones_v.at[pl.ds(0, CH)], acc_sh.at[idx_v.at[g]], sem
    ).wait()
    return 0

  lax.fori_loop(0, NCH, drain, 0)
  plsc.subcore_barrier()
  pltpu.sync_copy(
      acc_sh.at[pl.ds(s * RPT, RPT)], deg_out.at[c, pl.ds(s * RPT, RPT)]
  )


@functools.cache
def _deg_kernel():
  return pl.kernel(
      _deg_body,
      out_type=jax.ShapeDtypeStruct((NC, N, DEGL), jnp.float32),
      mesh=_sc_mesh(),
      scratch_types=[
          pltpu.VMEM((NCH, CH), jnp.int32),
          pltpu.VMEM((RPT, DEGL), jnp.float32),
          pltpu.VMEM_SHARED((NPAD, DEGL), jnp.float32),
          pltpu.SemaphoreType.DMA,
      ],
      compiler_params=pltpu.CompilerParams(use_tc_tiling_on_sc=False),
  )


def _agg_body(h2_hbm, ei_hbm, s_out, src_v, dst_v, rows0, rows1,
              rows2, fb_v, acc_sh, gs0, gs1, gs2, ss0, ss1, ss2):
  c = lax.axis_index("c")
  s = lax.axis_index("s")
  w = c * NS + s

  def gather(g, buf, sem):
    pltpu.async_copy(h2_hbm.at[src_v.at[g]], buf, sem)

  def gather_wait(g, buf, sem):
    pltpu.make_async_copy(h2_hbm.at[src_v.at[g]], buf, sem).wait()

  def scatter(g, buf, sem):
    pltpu.async_copy(buf, acc_sh.at[dst_v.at[g]], sem, add=True)

  def scatter_wait(g, buf, sem):
    pltpu.make_async_copy(buf, acc_sh.at[dst_v.at[g]], sem).wait()

  _zero_rows(rows0, CH, D, jnp.bfloat16)
  for i in range(RPT // CH):
    pltpu.sync_copy(rows0, acc_sh.at[pl.ds(s * RPT + i * CH, CH)])
  if RPT % CH:
    pltpu.sync_copy(
        rows0.at[pl.ds(0, RPT % CH)],
        acc_sh.at[pl.ds(s * RPT + (RPT // CH) * CH, RPT % CH)],
    )
  pltpu.sync_copy(ei_hbm.at[0, w], src_v)
  pltpu.sync_copy(ei_hbm.at[1, w], dst_v)
  plsc.subcore_barrier()

  gather(0, rows0, gs0)
  gather(1, rows1, gs1)
  gather(2, rows2, gs2)

  def ring(k, _):
    g = 3 * k
    gather_wait(g, rows0, gs0)
    scatter(g, rows0, ss0)
    gather_wait(g + 1, rows1, gs1)
    scatter(g + 1, rows1, ss1)
    gather_wait(g + 2, rows2, gs2)
    scatter(g + 2, rows2, ss2)
    scatter_wait(g, rows0, ss0)
    gather(g + 3, rows0, gs0)
    scatter_wait(g + 1, rows1, ss1)
    gather(g + 4, rows1, gs1)
    scatter_wait(g + 2, rows2, ss2)
    gather(g + 5, rows2, gs2)
    return 0

  lax.fori_loop(0, (NCH - 5) // 3, ring, 0)
  g = NCH - 5
  gather_wait(g, rows0, gs0)
  scatter(g, rows0, ss0)
  gather_wait(g + 1, rows1, gs1)
  scatter(g + 1, rows1, ss1)
  gather_wait(g + 2, rows2, gs2)
  scatter(g + 2, rows2, ss2)
  scatter_wait(g, rows0, ss0)
  gather(g + 3, rows0, gs0)
  scatter_wait(g + 1, rows1, ss1)
  gather(g + 4, rows1, gs1)
  gather_wait(g + 3, rows0, gs0)
  scatter(g + 3, rows0, ss0)
  gather_wait(g + 4, rows1, gs1)
  scatter(g + 4, rows1, ss1)
  scatter_wait(g + 2, rows2, ss2)
  scatter_wait(g + 3, rows0, ss0)
  scatter_wait(g + 4, rows1, ss1)
  plsc.subcore_barrier()

  cw = RPT // 5
  for i in range(5):
    r0 = s * RPT + i * cw
    pltpu.sync_copy(acc_sh.at[pl.ds(r0, cw)], rows0.at[pl.ds(0, cw)])

    def conv(r, _):
      for j in range(D // 32):
        a, bvals = plsc.unpack(
            rows0[r, pl.ds(j * 32, 32)], format=plsc.PackFormat.INTERLEAVED
        )
        fb_v[r, pl.ds(j * 32, 16)] = a
        fb_v[r, pl.ds(j * 32 + 16, 16)] = bvals
      return 0

    lax.fori_loop(0, cw, conv, 0)
    pltpu.sync_copy(fb_v, s_out.at[c, pl.ds(r0, cw)])


@functools.cache
def _agg_kernel():
  return pl.kernel(
      _agg_body,
      out_type=jax.ShapeDtypeStruct((NC, N, D), jnp.float32),
      mesh=_sc_mesh(),
      scratch_types=[
          pltpu.VMEM((NCH, CH), jnp.int32),
          pltpu.VMEM((NCH, CH), jnp.int32),
          pltpu.VMEM((CH, D), jnp.bfloat16),
          pltpu.VMEM((CH, D), jnp.bfloat16),
          pltpu.VMEM((CH, D), jnp.bfloat16),
          pltpu.VMEM((N // NS // 5, D), jnp.float32),
          pltpu.VMEM_SHARED((NPAD, D), jnp.bfloat16),
          pltpu.SemaphoreType.DMA,
          pltpu.SemaphoreType.DMA,
          pltpu.SemaphoreType.DMA,
          pltpu.SemaphoreType.DMA,
          pltpu.SemaphoreType.DMA,
          pltpu.SemaphoreType.DMA,
      ],
      compiler_params=pltpu.CompilerParams(
          use_tc_tiling_on_sc=False, needs_layout_passes=False
      ),
  )


BM = 1000


def _h2_body(x_ref, w_ref, degp_ref, h2_ref, dis_ref):
  deg = degp_ref[0] + degp_ref[1] + 1.0
  dis = lax.rsqrt(deg)
  h = jnp.dot(x_ref[...], w_ref[...], preferred_element_type=jnp.float32)
  h2_ref[...] = (h * dis[:, :1]).astype(jnp.bfloat16)
  dis_ref[...] = dis


@functools.cache
def _h2_kernel():
  return pl.pallas_call(
      _h2_body,
      grid=(N // BM,),
      in_specs=[
          pl.BlockSpec((BM, D), lambda i: (i, 0)),
          pl.BlockSpec((D, D), lambda i: (0, 0)),
          pl.BlockSpec((NC, BM, DEGL), lambda i: (0, i, 0)),
      ],
      out_specs=[
          pl.BlockSpec((BM, D), lambda i: (i, 0)),
          pl.BlockSpec((BM, DEGL), lambda i: (i, 0)),
      ],
      out_shape=[
          jax.ShapeDtypeStruct((N, D), jnp.bfloat16),
          jax.ShapeDtypeStruct((N, DEGL), jnp.float32),
      ],
  )


def _final_body(sp_ref, h2_ref, dis_ref, x_ref, b_ref, g_ref, be_ref, o_ref):
  ssum = sp_ref[0] + sp_ref[1]
  ssum = (
      ssum.reshape(BM, D // 32, 2, 16)
      .transpose(0, 1, 3, 2)
      .reshape(BM, D)
  )
  g = dis_ref[:, :1] * (ssum + h2_ref[...].astype(jnp.float32)) + b_ref[...]
  mu = jnp.mean(g, axis=-1, keepdims=True)
  var = jnp.mean((g - mu) ** 2, axis=-1, keepdims=True)
  ln = (g - mu) / jnp.sqrt(var + 1e-5) * g_ref[...] + be_ref[...]
  o_ref[...] = jnp.maximum(ln, 0.0) + x_ref[...]


@functools.cache
def _final_kernel():
  return pl.pallas_call(
      _final_body,
      grid=(N // BM,),
      in_specs=[
          pl.BlockSpec((NC, BM, D), lambda i: (0, i, 0)),
          pl.BlockSpec((BM, D), lambda i: (i, 0)),
          pl.BlockSpec((BM, DEGL), lambda i: (i, 0)),
          pl.BlockSpec((BM, D), lambda i: (i, 0)),
          pl.BlockSpec((1, D), lambda i: (0, 0)),
          pl.BlockSpec((1, D), lambda i: (0, 0)),
          pl.BlockSpec((1, D), lambda i: (0, 0)),
      ],
      out_specs=pl.BlockSpec((BM, D), lambda i: (i, 0)),
      out_shape=jax.ShapeDtypeStruct((N, D), jnp.float32),
  )


@jax.jit
def kernel(x, edge_index, W, b, ln_gamma, ln_beta):
  ei = edge_index.astype(jnp.int32).reshape(2, NW, EPW)
  fill_src = (
      jnp.arange(NW, dtype=jnp.int32)[:, None] * 311
      + jnp.arange(PAD, dtype=jnp.int32)[None, :] * 37
  ) % N
  srcp = jnp.concatenate([ei[0], fill_src], axis=1)
  fill_dst = jnp.broadcast_to(
      N + (jnp.arange(NW, dtype=jnp.int32) % NS)[:, None], (NW, PAD)
  )
  dstp = jnp.concatenate([ei[1], fill_dst], axis=1)
  ei_p = jnp.stack([srcp, dstp]).reshape(2, NW, NCH, CH)

  deg_part = _deg_kernel()(ei_p)
  h2, dis = _h2_kernel()(x, W, deg_part)
  s_part = _agg_kernel()(h2, ei_p)
  return _final_kernel()(
      s_part, h2, dis, x,
      b.reshape(1, D), ln_gamma.reshape(1, D), ln_beta.reshape(1, D),
  )


# speedup vs baseline: 1.9994x; 1.9994x over previous
import functools

import jax
import jax.numpy as jnp
from jax import lax
from jax.experimental import pallas as pl
from jax.experimental.pallas import tpu as pltpu
from jax.experimental.pallas import tpu_sc as plsc

N = 10000
D = 128
E = 320000

NC = 2
NS = 16
NW = NC * NS
EPW = E // NW
CH = 128
NCH = 80
PAD = NCH * CH - EPW
NPAD = N + NS
RPT = N // NS
DEGL = 16


def _zero_rows(buf, nrows, ncols, dtype=jnp.float32):
  lanes = 32 if dtype == jnp.bfloat16 else 16
  z = jnp.zeros((lanes,), dtype)

  def body(r, _):
    for j in range(ncols // lanes):
      buf[r, pl.ds(j * lanes, lanes)] = z
    return 0

  lax.fori_loop(0, nrows, body, 0)


def _sc_mesh():
  return plsc.VectorSubcoreMesh(
      core_axis_name="c", subcore_axis_name="s", num_cores=NC, num_subcores=NS
  )


def _deg_body(ei_hbm, deg_out, idx_v, ones_v, acc_sh, sem):
  c = lax.axis_index("c")
  s = lax.axis_index("s")
  w = c * NS + s

  _zero_rows(ones_v, RPT, DEGL)
  pltpu.sync_copy(ones_v, acc_sh.at[pl.ds(s * RPT, RPT)])
  one = jnp.full((16,), 1.0, jnp.float32)

  def fill(r, _):
    ones_v[r, :] = one
    return 0

  lax.fori_loop(0, CH, fill, 0)
  pltpu.sync_copy(ei_hbm.at[1, w], idx_v)
  plsc.subcore_barrier()

  def chunk(g, _):
    pltpu.async_copy(
        ones_v.at[pl.ds(0, CH)], acc_sh.at[idx_v.at[g]], sem, add=True
    )
    return 0

  lax.fori_loop(0, NCH, chunk, 0)

  def drain(g, _):
    pltpu.make_async_copy(
        ones_v.at[pl.ds(0, CH)], acc_sh.at[idx_v.at[g]], sem
    ).wait()
    return 0

  lax.fori_loop(0, NCH, drain, 0)
  plsc.subcore_barrier()
  pltpu.sync_copy(
      acc_sh.at[pl.ds(s * RPT, RPT)], deg_out.at[c, pl.ds(s * RPT, RPT)]
  )


@functools.cache
def _deg_kernel():
  return pl.kernel(
      _deg_body,
      out_type=jax.ShapeDtypeStruct((NC, N, DEGL), jnp.float32),
      mesh=_sc_mesh(),
      scratch_types=[
          pltpu.VMEM((NCH, CH), jnp.int32),
          pltpu.VMEM((RPT, DEGL), jnp.float32),
          pltpu.VMEM_SHARED((NPAD, DEGL), jnp.float32),
          pltpu.SemaphoreType.DMA,
      ],
      compiler_params=pltpu.CompilerParams(use_tc_tiling_on_sc=False),
  )


def _agg_body(h2_hbm, ei_hbm, s_out, src_v, dst_v, rows0, rows1,
              rows2, fb_v, acc_sh, gs0, gs1, gs2, ss0, ss1, ss2):
  c = lax.axis_index("c")
  s = lax.axis_index("s")
  w = c * NS + s

  def gather(g, buf, sem):
    pltpu.async_copy(h2_hbm.at[src_v.at[g]], buf, sem)

  def gather_wait(g, buf, sem):
    pltpu.make_async_copy(h2_hbm.at[src_v.at[g]], buf, sem).wait()

  def scatter(g, buf, sem):
    pltpu.async_copy(buf, acc_sh.at[dst_v.at[g]], sem, add=True)

  def scatter_wait(g, buf, sem):
    pltpu.make_async_copy(buf, acc_sh.at[dst_v.at[g]], sem).wait()

  _zero_rows(rows0, CH, D, jnp.bfloat16)
  for i in range(RPT // CH):
    pltpu.sync_copy(rows0, acc_sh.at[pl.ds(s * RPT + i * CH, CH)])
  if RPT % CH:
    pltpu.sync_copy(
        rows0.at[pl.ds(0, RPT % CH)],
        acc_sh.at[pl.ds(s * RPT + (RPT // CH) * CH, RPT % CH)],
    )
  pltpu.sync_copy(ei_hbm.at[0, w], src_v)
  pltpu.sync_copy(ei_hbm.at[1, w], dst_v)
  plsc.subcore_barrier()

  gather(0, rows0, gs0)
  gather(1, rows1, gs1)
  gather(2, rows2, gs2)

  def ring(k, _):
    g = 3 * k
    gather_wait(g, rows0, gs0)
    scatter(g, rows0, ss0)
    gather_wait(g + 1, rows1, gs1)
    scatter(g + 1, rows1, ss1)
    gather_wait(g + 2, rows2, gs2)
    scatter(g + 2, rows2, ss2)
    scatter_wait(g, rows0, ss0)
    gather(g + 3, rows0, gs0)
    scatter_wait(g + 1, rows1, ss1)
    gather(g + 4, rows1, gs1)
    scatter_wait(g + 2, rows2, ss2)
    gather(g + 5, rows2, gs2)
    return 0

  lax.fori_loop(0, (NCH - 5) // 3, ring, 0)
  g = NCH - 5
  gather_wait(g, rows0, gs0)
  scatter(g, rows0, ss0)
  gather_wait(g + 1, rows1, gs1)
  scatter(g + 1, rows1, ss1)
  gather_wait(g + 2, rows2, gs2)
  scatter(g + 2, rows2, ss2)
  scatter_wait(g, rows0, ss0)
  gather(g + 3, rows0, gs0)
  scatter_wait(g + 1, rows1, ss1)
  gather(g + 4, rows1, gs1)
  gather_wait(g + 3, rows0, gs0)
  scatter(g + 3, rows0, ss0)
  gather_wait(g + 4, rows1, gs1)
  scatter(g + 4, rows1, ss1)
  scatter_wait(g + 2, rows2, ss2)
  scatter_wait(g + 3, rows0, ss0)
  scatter_wait(g + 4, rows1, ss1)
  plsc.subcore_barrier()

  cw = RPT // 5
  evens = lax.iota(jnp.int32, 16) * 2
  for i in range(5):
    r0 = s * RPT + i * cw
    pltpu.sync_copy(acc_sh.at[pl.ds(r0, cw)], rows0.at[pl.ds(0, cw)])

    def conv(r, _):
      rvec = jnp.zeros((16,), jnp.int32) + r
      for j in range(D // 32):
        a, bvals = plsc.unpack(
            rows0[r, pl.ds(j * 32, 32)], format=plsc.PackFormat.INTERLEAVED
        )
        plsc.store_scatter(fb_v, [rvec, evens + (j * 32)], a)
        plsc.store_scatter(fb_v, [rvec, evens + (j * 32 + 1)], bvals)
      return 0

    lax.fori_loop(0, cw, conv, 0)
    pltpu.sync_copy(fb_v, s_out.at[c, pl.ds(r0, cw)])


@functools.cache
def _agg_kernel():
  return pl.kernel(
      _agg_body,
      out_type=jax.ShapeDtypeStruct((NC, N, D), jnp.float32),
      mesh=_sc_mesh(),
      scratch_types=[
          pltpu.VMEM((NCH, CH), jnp.int32),
          pltpu.VMEM((NCH, CH), jnp.int32),
          pltpu.VMEM((CH, D), jnp.bfloat16),
          pltpu.VMEM((CH, D), jnp.bfloat16),
          pltpu.VMEM((CH, D), jnp.bfloat16),
          pltpu.VMEM((N // NS // 5, D), jnp.float32),
          pltpu.VMEM_SHARED((NPAD, D), jnp.bfloat16),
          pltpu.SemaphoreType.DMA,
          pltpu.SemaphoreType.DMA,
          pltpu.SemaphoreType.DMA,
          pltpu.SemaphoreType.DMA,
          pltpu.SemaphoreType.DMA,
          pltpu.SemaphoreType.DMA,
      ],
      compiler_params=pltpu.CompilerParams(
          use_tc_tiling_on_sc=False, needs_layout_passes=False
      ),
  )


BM = 1000


def _h2_body(x_ref, w_ref, degp_ref, h2_ref, dis_ref):
  deg = degp_ref[0] + degp_ref[1] + 1.0
  dis = lax.rsqrt(deg)
  h = jnp.dot(x_ref[...], w_ref[...], preferred_element_type=jnp.float32)
  h2_ref[...] = (h * dis[:, :1]).astype(jnp.bfloat16)
  dis_ref[...] = dis


@functools.cache
def _h2_kernel():
  return pl.pallas_call(
      _h2_body,
      grid=(N // BM,),
      in_specs=[
          pl.BlockSpec((BM, D), lambda i: (i, 0)),
          pl.BlockSpec((D, D), lambda i: (0, 0)),
          pl.BlockSpec((NC, BM, DEGL), lambda i: (0, i, 0)),
      ],
      out_specs=[
          pl.BlockSpec((BM, D), lambda i: (i, 0)),
          pl.BlockSpec((BM, DEGL), lambda i: (i, 0)),
      ],
      out_shape=[
          jax.ShapeDtypeStruct((N, D), jnp.bfloat16),
          jax.ShapeDtypeStruct((N, DEGL), jnp.float32),
      ],
  )


def _final_body(sp_ref, h2_ref, dis_ref, x_ref, b_ref, g_ref, be_ref, o_ref):
  ssum = sp_ref[0] + sp_ref[1]
  g = dis_ref[:, :1] * (ssum + h2_ref[...].astype(jnp.float32)) + b_ref[...]
  mu = jnp.mean(g, axis=-1, keepdims=True)
  var = jnp.mean((g - mu) ** 2, axis=-1, keepdims=True)
  ln = (g - mu) / jnp.sqrt(var + 1e-5) * g_ref[...] + be_ref[...]
  o_ref[...] = jnp.maximum(ln, 0.0) + x_ref[...]


@functools.cache
def _final_kernel():
  return pl.pallas_call(
      _final_body,
      grid=(N // BM,),
      in_specs=[
          pl.BlockSpec((NC, BM, D), lambda i: (0, i, 0)),
          pl.BlockSpec((BM, D), lambda i: (i, 0)),
          pl.BlockSpec((BM, DEGL), lambda i: (i, 0)),
          pl.BlockSpec((BM, D), lambda i: (i, 0)),
          pl.BlockSpec((1, D), lambda i: (0, 0)),
          pl.BlockSpec((1, D), lambda i: (0, 0)),
          pl.BlockSpec((1, D), lambda i: (0, 0)),
      ],
      out_specs=pl.BlockSpec((BM, D), lambda i: (i, 0)),
      out_shape=jax.ShapeDtypeStruct((N, D), jnp.float32),
  )


@jax.jit
def kernel(x, edge_index, W, b, ln_gamma, ln_beta):
  ei = edge_index.astype(jnp.int32).reshape(2, NW, EPW)
  fill_src = (
      jnp.arange(NW, dtype=jnp.int32)[:, None] * 311
      + jnp.arange(PAD, dtype=jnp.int32)[None, :] * 37
  ) % N
  srcp = jnp.concatenate([ei[0], fill_src], axis=1)
  fill_dst = jnp.broadcast_to(
      N + (jnp.arange(NW, dtype=jnp.int32) % NS)[:, None], (NW, PAD)
  )
  dstp = jnp.concatenate([ei[1], fill_dst], axis=1)
  ei_p = jnp.stack([srcp, dstp]).reshape(2, NW, NCH, CH)

  deg_part = _deg_kernel()(ei_p)
  h2, dis = _h2_kernel()(x, W, deg_part)
  s_part = _agg_kernel()(h2, ei_p)
  return _final_kernel()(
      s_part, h2, dis, x,
      b.reshape(1, D), ln_gamma.reshape(1, D), ln_beta.reshape(1, D),
  )


# speedup vs baseline: 2.0122x; 1.0064x over previous
import functools

import jax
import jax.numpy as jnp
from jax import lax
from jax.experimental import pallas as pl
from jax.experimental.pallas import tpu as pltpu
from jax.experimental.pallas import tpu_sc as plsc

N = 10000
D = 128
E = 320000

NC = 2
NS = 16
NW = NC * NS
EPW = E // NW
CH = 128
NCH = 80
PAD = NCH * CH - EPW
NPAD = N + NS
RPT = N // NS
DEGL = 16


def _zero_rows(buf, nrows, ncols, dtype=jnp.float32):
  lanes = 32 if dtype == jnp.bfloat16 else 16
  z = jnp.zeros((lanes,), dtype)

  def body(r, _):
    for j in range(ncols // lanes):
      buf[r, pl.ds(j * lanes, lanes)] = z
    return 0

  lax.fori_loop(0, nrows, body, 0)


def _sc_mesh():
  return plsc.VectorSubcoreMesh(
      core_axis_name="c", subcore_axis_name="s", num_cores=NC, num_subcores=NS
  )


def _deg_body(ei_hbm, deg_out, idx_v, ones_v, acc_sh, sem):
  c = lax.axis_index("c")
  s = lax.axis_index("s")
  w = c * NS + s

  _zero_rows(ones_v, RPT, DEGL)
  pltpu.sync_copy(ones_v, acc_sh.at[pl.ds(s * RPT, RPT)])
  one = jnp.full((16,), 1.0, jnp.float32)

  def fill(r, _):
    ones_v[r, :] = one
    return 0

  lax.fori_loop(0, CH, fill, 0)
  pltpu.sync_copy(ei_hbm.at[1, w], idx_v)
  plsc.subcore_barrier()

  def chunk(g, _):
    pltpu.async_copy(
        ones_v.at[pl.ds(0, CH)], acc_sh.at[idx_v.at[g]], sem, add=True
    )
    return 0

  lax.fori_loop(0, NCH, chunk, 0)

  def drain(g, _):
    pltpu.make_async_copy(
        ones_v.at[pl.ds(0, CH)], acc_sh.at[idx_v.at[g]], sem
    ).wait()
    return 0

  lax.fori_loop(0, NCH, drain, 0)
  plsc.subcore_barrier()
  pltpu.sync_copy(
      acc_sh.at[pl.ds(s * RPT, RPT)], deg_out.at[c, pl.ds(s * RPT, RPT)]
  )


@functools.cache
def _deg_kernel():
  return pl.kernel(
      _deg_body,
      out_type=jax.ShapeDtypeStruct((NC, N, DEGL), jnp.float32),
      mesh=_sc_mesh(),
      scratch_types=[
          pltpu.VMEM((NCH, CH), jnp.int32),
          pltpu.VMEM((RPT, DEGL), jnp.float32),
          pltpu.VMEM_SHARED((NPAD, DEGL), jnp.float32),
          pltpu.SemaphoreType.DMA,
      ],
      compiler_params=pltpu.CompilerParams(use_tc_tiling_on_sc=False),
  )


def _agg_body(h2_hbm, ei_hbm, s_out, src_v, dst_v, rows0, rows1,
              rows2, fb_v, acc_sh, gs0, gs1, gs2, ss0, ss1, ss2):
  c = lax.axis_index("c")
  s = lax.axis_index("s")
  w = c * NS + s

  def gather(g, buf, sem):
    pltpu.async_copy(h2_hbm.at[src_v.at[g]], buf, sem)

  def gather_wait(g, buf, sem):
    pltpu.make_async_copy(h2_hbm.at[src_v.at[g]], buf, sem).wait()

  def scatter(g, buf, sem):
    pltpu.async_copy(buf, acc_sh.at[dst_v.at[g]], sem, add=True)

  def scatter_wait(g, buf, sem):
    pltpu.make_async_copy(buf, acc_sh.at[dst_v.at[g]], sem).wait()

  @pl.when(c == 0)
  def _():
    pltpu.sync_copy(
        h2_hbm.at[pl.ds(s * RPT, RPT)], acc_sh.at[pl.ds(s * RPT, RPT)]
    )

  @pl.when(c != 0)
  def _():
    _zero_rows(rows0, CH, D, jnp.bfloat16)
    for i in range(RPT // CH):
      pltpu.sync_copy(rows0, acc_sh.at[pl.ds(s * RPT + i * CH, CH)])
    if RPT % CH:
      pltpu.sync_copy(
          rows0.at[pl.ds(0, RPT % CH)],
          acc_sh.at[pl.ds(s * RPT + (RPT // CH) * CH, RPT % CH)],
      )

  pltpu.sync_copy(ei_hbm.at[0, w], src_v)
  pltpu.sync_copy(ei_hbm.at[1, w], dst_v)
  plsc.subcore_barrier()

  gather(0, rows0, gs0)
  gather(1, rows1, gs1)
  gather(2, rows2, gs2)

  def ring(k, _):
    g = 3 * k
    gather_wait(g, rows0, gs0)
    scatter(g, rows0, ss0)
    gather_wait(g + 1, rows1, gs1)
    scatter(g + 1, rows1, ss1)
    gather_wait(g + 2, rows2, gs2)
    scatter(g + 2, rows2, ss2)
    scatter_wait(g, rows0, ss0)
    gather(g + 3, rows0, gs0)
    scatter_wait(g + 1, rows1, ss1)
    gather(g + 4, rows1, gs1)
    scatter_wait(g + 2, rows2, ss2)
    gather(g + 5, rows2, gs2)
    return 0

  lax.fori_loop(0, (NCH - 5) // 3, ring, 0)
  g = NCH - 5
  gather_wait(g, rows0, gs0)
  scatter(g, rows0, ss0)
  gather_wait(g + 1, rows1, gs1)
  scatter(g + 1, rows1, ss1)
  gather_wait(g + 2, rows2, gs2)
  scatter(g + 2, rows2, ss2)
  scatter_wait(g, rows0, ss0)
  gather(g + 3, rows0, gs0)
  scatter_wait(g + 1, rows1, ss1)
  gather(g + 4, rows1, gs1)
  gather_wait(g + 3, rows0, gs0)
  scatter(g + 3, rows0, ss0)
  gather_wait(g + 4, rows1, gs1)
  scatter(g + 4, rows1, ss1)
  scatter_wait(g + 2, rows2, ss2)
  scatter_wait(g + 3, rows0, ss0)
  scatter_wait(g + 4, rows1, ss1)
  plsc.subcore_barrier()

  cw = RPT // 5
  evens = lax.iota(jnp.int32, 16) * 2
  for i in range(5):
    r0 = s * RPT + i * cw
    pltpu.sync_copy(acc_sh.at[pl.ds(r0, cw)], rows0.at[pl.ds(0, cw)])

    def conv(r, _):
      rvec = jnp.zeros((16,), jnp.int32) + r
      for j in range(D // 32):
        a, bvals = plsc.unpack(
            rows0[r, pl.ds(j * 32, 32)], format=plsc.PackFormat.INTERLEAVED
        )
        plsc.store_scatter(fb_v, [rvec, evens + (j * 32)], a)
        plsc.store_scatter(fb_v, [rvec, evens + (j * 32 + 1)], bvals)
      return 0

    lax.fori_loop(0, cw, conv, 0)
    pltpu.sync_copy(fb_v, s_out.at[c, pl.ds(r0, cw)])


@functools.cache
def _agg_kernel():
  return pl.kernel(
      _agg_body,
      out_type=jax.ShapeDtypeStruct((NC, N, D), jnp.float32),
      mesh=_sc_mesh(),
      scratch_types=[
          pltpu.VMEM((NCH, CH), jnp.int32),
          pltpu.VMEM((NCH, CH), jnp.int32),
          pltpu.VMEM((CH, D), jnp.bfloat16),
          pltpu.VMEM((CH, D), jnp.bfloat16),
          pltpu.VMEM((CH, D), jnp.bfloat16),
          pltpu.VMEM((N // NS // 5, D), jnp.float32),
          pltpu.VMEM_SHARED((NPAD, D), jnp.bfloat16),
          pltpu.SemaphoreType.DMA,
          pltpu.SemaphoreType.DMA,
          pltpu.SemaphoreType.DMA,
          pltpu.SemaphoreType.DMA,
          pltpu.SemaphoreType.DMA,
          pltpu.SemaphoreType.DMA,
      ],
      compiler_params=pltpu.CompilerParams(
          use_tc_tiling_on_sc=False, needs_layout_passes=False
      ),
  )


BM = 1000


def _h2_body(x_ref, w_ref, degp_ref, h2_ref, dis_ref):
  deg = degp_ref[0] + degp_ref[1] + 1.0
  dis = lax.rsqrt(deg)
  h = jnp.dot(x_ref[...], w_ref[...], preferred_element_type=jnp.float32)
  h2_ref[...] = (h * dis[:, :1]).astype(jnp.bfloat16)
  dis_ref[...] = dis


@functools.cache
def _h2_kernel():
  return pl.pallas_call(
      _h2_body,
      grid=(N // BM,),
      in_specs=[
          pl.BlockSpec((BM, D), lambda i: (i, 0)),
          pl.BlockSpec((D, D), lambda i: (0, 0)),
          pl.BlockSpec((NC, BM, DEGL), lambda i: (0, i, 0)),
      ],
      out_specs=[
          pl.BlockSpec((BM, D), lambda i: (i, 0)),
          pl.BlockSpec((BM, DEGL), lambda i: (i, 0)),
      ],
      out_shape=[
          jax.ShapeDtypeStruct((N, D), jnp.bfloat16),
          jax.ShapeDtypeStruct((N, DEGL), jnp.float32),
      ],
  )


def _final_body(sp_ref, dis_ref, x_ref, b_ref, g_ref, be_ref, o_ref):
  ssum = sp_ref[0] + sp_ref[1]
  g = dis_ref[:, :1] * ssum + b_ref[...]
  mu = jnp.mean(g, axis=-1, keepdims=True)
  var = jnp.mean((g - mu) ** 2, axis=-1, keepdims=True)
  ln = (g - mu) / jnp.sqrt(var + 1e-5) * g_ref[...] + be_ref[...]
  o_ref[...] = jnp.maximum(ln, 0.0) + x_ref[...]


@functools.cache
def _final_kernel():
  return pl.pallas_call(
      _final_body,
      grid=(N // BM,),
      in_specs=[
          pl.BlockSpec((NC, BM, D), lambda i: (0, i, 0)),
          pl.BlockSpec((BM, DEGL), lambda i: (i, 0)),
          pl.BlockSpec((BM, D), lambda i: (i, 0)),
          pl.BlockSpec((1, D), lambda i: (0, 0)),
          pl.BlockSpec((1, D), lambda i: (0, 0)),
          pl.BlockSpec((1, D), lambda i: (0, 0)),
      ],
      out_specs=pl.BlockSpec((BM, D), lambda i: (i, 0)),
      out_shape=jax.ShapeDtypeStruct((N, D), jnp.float32),
  )


@jax.jit
def kernel(x, edge_index, W, b, ln_gamma, ln_beta):
  ei = edge_index.astype(jnp.int32).reshape(2, NW, EPW)
  fill_src = (
      jnp.arange(NW, dtype=jnp.int32)[:, None] * 311
      + jnp.arange(PAD, dtype=jnp.int32)[None, :] * 37
  ) % N
  srcp = jnp.concatenate([ei[0], fill_src], axis=1)
  fill_dst = jnp.broadcast_to(
      N + (jnp.arange(NW, dtype=jnp.int32) % NS)[:, None], (NW, PAD)
  )
  dstp = jnp.concatenate([ei[1], fill_dst], axis=1)
  ei_p = jnp.stack([srcp, dstp]).reshape(2, NW, NCH, CH)

  deg_part = _deg_kernel()(ei_p)
  h2, dis = _h2_kernel()(x, W, deg_part)
  s_part = _agg_kernel()(h2, ei_p)
  return _final_kernel()(
      s_part, dis, x,
      b.reshape(1, D), ln_gamma.reshape(1, D), ln_beta.reshape(1, D),
  )
